# Initial kernel scaffold; baseline (speedup 1.0000x reference)
import functools

import jax
import jax.numpy as jnp
from jax import lax
from jax.experimental import pallas as pl
from jax.experimental.pallas import tpu as pltpu
from jax.experimental.pallas import tpu_sc as plsc

N = 10000
E = 160000
D_IN = 256
D_HID = 16
N_CLS = 3

NC = 2
NS = 16
NW = NC * NS
CHUNK = 128
N_PAD = 10240
EPW = 5120
NCHUNK = EPW // CHUNK
ROWS_PER_TILE = N_PAD // NS

_MESH = plsc.VectorSubcoreMesh(core_axis_name="c", subcore_axis_name="s")


@functools.partial(
    pl.kernel,
    out_type=jax.ShapeDtypeStruct((NC, N_PAD, D_HID), jnp.float32),
    mesh=_MESH,
    scratch_types=[
        pltpu.VMEM((NCHUNK, CHUNK), jnp.int32),
        pltpu.VMEM((CHUNK, D_HID), jnp.float32),
        pltpu.VMEM_SHARED((N_PAD, D_HID), jnp.float32),
    ],
)
def _sc_degree(dst_hbm, ones_hbm, zeros_hbm, out_hbm, dstv, onesv, acc):
    cid = lax.axis_index("c")
    sid = lax.axis_index("s")
    wid = cid * NS + sid
    r0 = sid * ROWS_PER_TILE
    pltpu.sync_copy(zeros_hbm.at[pl.ds(r0, ROWS_PER_TILE)],
                    acc.at[pl.ds(r0, ROWS_PER_TILE)])
    pltpu.sync_copy(ones_hbm, onesv)
    pltpu.sync_copy(dst_hbm.at[wid], dstv)
    plsc.subcore_barrier()

    @pl.loop(0, NCHUNK)
    def _(j):
        pltpu.sync_copy(onesv, acc.at[dstv.at[j]], add=True)

    plsc.subcore_barrier()
    pltpu.sync_copy(acc.at[pl.ds(r0, ROWS_PER_TILE)],
                    out_hbm.at[cid, pl.ds(r0, ROWS_PER_TILE)])


@functools.partial(
    pl.kernel,
    out_type=jax.ShapeDtypeStruct((NC, N_PAD, D_HID), jnp.float32),
    mesh=_MESH,
    scratch_types=[
        pltpu.VMEM((NCHUNK, CHUNK), jnp.int32),
        pltpu.VMEM((NCHUNK, CHUNK), jnp.int32),
        pltpu.VMEM((CHUNK, D_HID), jnp.float32),
        pltpu.VMEM_SHARED((N_PAD, D_HID), jnp.float32),
    ],
)
def _sc_spmm(g_hbm, src_hbm, dst_hbm, zeros_hbm, out_hbm, srcv, dstv, rows, acc):
    cid = lax.axis_index("c")
    sid = lax.axis_index("s")
    wid = cid * NS + sid
    r0 = sid * ROWS_PER_TILE
    pltpu.sync_copy(zeros_hbm.at[pl.ds(r0, ROWS_PER_TILE)],
                    acc.at[pl.ds(r0, ROWS_PER_TILE)])
    pltpu.sync_copy(src_hbm.at[wid], srcv)
    pltpu.sync_copy(dst_hbm.at[wid], dstv)
    plsc.subcore_barrier()

    @pl.loop(0, NCHUNK)
    def _(j):
        pltpu.sync_copy(g_hbm.at[srcv.at[j]], rows)
        pltpu.sync_copy(rows, acc.at[dstv.at[j]], add=True)

    plsc.subcore_barrier()
    pltpu.sync_copy(acc.at[pl.ds(r0, ROWS_PER_TILE)],
                    out_hbm.at[cid, pl.ds(r0, ROWS_PER_TILE)])


def _tc_mm_body(x_ref, w_ref, o_ref):
    o_ref[...] = jnp.dot(x_ref[...], w_ref[...],
                         preferred_element_type=jnp.float32)


def _tc_scale_body(degp_ref, h1_ref, g_ref, dinv_ref):
    deg = degp_ref[0, :, 0:1] + degp_ref[1, :, 0:1] + 1.0
    dinv = lax.rsqrt(deg)
    row = lax.broadcasted_iota(jnp.int32, (N_PAD, 1), 0)
    dinv = jnp.where(row < N, dinv, 0.0)
    dinvb = jnp.broadcast_to(dinv, (N_PAD, D_HID))
    dinv_ref[...] = dinvb
    g_ref[...] = dinvb * h1_ref[...]


def _tc_mid_body(s1p_ref, g_ref, dinv_ref, b1_ref, u_ref):
    s = s1p_ref[0] + s1p_ref[1] + g_ref[...]
    z = jnp.maximum(dinv_ref[...] * s + b1_ref[...], 0.0)
    u_ref[...] = dinv_ref[...] * z


def _tc_final_body(s2p_ref, u_ref, dinv_ref, w2_ref, b2_ref, o_ref):
    y = dinv_ref[...] * (s2p_ref[0] + s2p_ref[1] + u_ref[...])
    logits = jnp.dot(y, w2_ref[...], preferred_element_type=jnp.float32)
    logits = logits[:N] + b2_ref[...]
    m = jnp.max(logits, axis=1, keepdims=True)
    e = jnp.exp(logits - m)
    lse = m + jnp.log(jnp.sum(e, axis=1, keepdims=True))
    o_ref[...] = logits - lse


def _f32(shape):
    return jax.ShapeDtypeStruct(shape, jnp.float32)


def kernel(x, edge_index, W1, b1, W2, b2):
    src = edge_index[0]
    dst = edge_index[1]
    pad_e = NW * EPW - E
    src_p = jnp.concatenate(
        [src, jnp.full((pad_e,), N_PAD - 1, jnp.int32)]).reshape(NW, NCHUNK, CHUNK)
    dst_p = jnp.concatenate(
        [dst, jnp.full((pad_e,), N_PAD - 1, jnp.int32)]).reshape(NW, NCHUNK, CHUNK)
    x_pad = jnp.zeros((N_PAD, D_IN), jnp.float32).at[:N].set(x)
    zeros = jnp.zeros((N_PAD, D_HID), jnp.float32)
    ones_rows = jnp.ones((CHUNK, D_HID), jnp.float32)

    h1 = pl.pallas_call(_tc_mm_body, out_shape=_f32((N_PAD, D_HID)))(x_pad, W1)
    degp = _sc_degree(dst_p, ones_rows, zeros)
    g1, dinvb = pl.pallas_call(
        _tc_scale_body,
        out_shape=[_f32((N_PAD, D_HID)), _f32((N_PAD, D_HID))])(degp, h1)
    s1p = _sc_spmm(g1, src_p, dst_p, zeros)
    u = pl.pallas_call(_tc_mid_body, out_shape=_f32((N_PAD, D_HID)))(
        s1p, g1, dinvb, b1)
    s2p = _sc_spmm(u, src_p, dst_p, zeros)
    out = pl.pallas_call(_tc_final_body, out_shape=_f32((N, N_CLS)))(
        s2p, u, dinvb, W2, b2)
    return out


# speedup vs baseline: 20.2552x; 20.2552x over previous
import functools

import jax
import jax.numpy as jnp
from jax import lax
from jax.experimental import pallas as pl
from jax.experimental.pallas import tpu as pltpu
from jax.experimental.pallas import tpu_sc as plsc

N = 10000
E = 160000
D_IN = 256
D_HID = 16
N_CLS = 3

NC = 2
NS = 16
NW = NC * NS
CHUNK = 128
N_PAD = 10240
EPW = 5120
NCHUNK = EPW // CHUNK
ROWS_PER_TILE = N_PAD // NS


@functools.cache
def _sc_kernels():
    mesh = plsc.VectorSubcoreMesh(core_axis_name="c", subcore_axis_name="s")
    partials = jax.ShapeDtypeStruct((NC, N_PAD, D_HID), jnp.float32)
    cparams = pltpu.CompilerParams(use_tc_tiling_on_sc=False)

    @functools.partial(
        pl.kernel,
        out_type=partials,
        mesh=mesh,
        scratch_types=[
            pltpu.VMEM((NCHUNK, CHUNK), jnp.int32),
            pltpu.VMEM((CHUNK, D_HID), jnp.float32),
            pltpu.VMEM_SHARED((N_PAD, D_HID), jnp.float32),
        ],
        compiler_params=cparams,
    )
    def sc_degree(dst_hbm, ones_hbm, zeros_hbm, out_hbm, dstv, onesv, acc):
        cid = lax.axis_index("c")
        sid = lax.axis_index("s")
        wid = cid * NS + sid
        r0 = sid * ROWS_PER_TILE
        pltpu.sync_copy(zeros_hbm.at[pl.ds(r0, ROWS_PER_TILE)],
                        acc.at[pl.ds(r0, ROWS_PER_TILE)])
        pltpu.sync_copy(ones_hbm, onesv)
        pltpu.sync_copy(dst_hbm.at[wid], dstv)
        plsc.subcore_barrier()

        @pl.loop(0, NCHUNK)
        def _(j):
            pltpu.sync_copy(onesv, acc.at[dstv.at[j]], add=True)

        plsc.subcore_barrier()
        pltpu.sync_copy(acc.at[pl.ds(r0, ROWS_PER_TILE)],
                        out_hbm.at[cid, pl.ds(r0, ROWS_PER_TILE)])

    @functools.partial(
        pl.kernel,
        out_type=partials,
        mesh=mesh,
        scratch_types=[
            pltpu.VMEM((NCHUNK, CHUNK), jnp.int32),
            pltpu.VMEM((NCHUNK, CHUNK), jnp.int32),
            pltpu.VMEM((CHUNK, D_HID), jnp.float32),
            pltpu.VMEM_SHARED((N_PAD, D_HID), jnp.float32),
        ],
        compiler_params=cparams,
    )
    def sc_spmm(g_hbm, src_hbm, dst_hbm, zeros_hbm, out_hbm,
                srcv, dstv, rows, acc):
        cid = lax.axis_index("c")
        sid = lax.axis_index("s")
        wid = cid * NS + sid
        r0 = sid * ROWS_PER_TILE
        pltpu.sync_copy(zeros_hbm.at[pl.ds(r0, ROWS_PER_TILE)],
                        acc.at[pl.ds(r0, ROWS_PER_TILE)])
        pltpu.sync_copy(src_hbm.at[wid], srcv)
        pltpu.sync_copy(dst_hbm.at[wid], dstv)
        plsc.subcore_barrier()

        @pl.loop(0, NCHUNK)
        def _(j):
            pltpu.sync_copy(g_hbm.at[srcv.at[j]], rows)
            pltpu.sync_copy(rows, acc.at[dstv.at[j]], add=True)

        plsc.subcore_barrier()
        pltpu.sync_copy(acc.at[pl.ds(r0, ROWS_PER_TILE)],
                        out_hbm.at[cid, pl.ds(r0, ROWS_PER_TILE)])

    return sc_degree, sc_spmm


def _tc_mm_body(x_ref, w_ref, o_ref):
    o_ref[...] = jnp.dot(x_ref[...], w_ref[...],
                         preferred_element_type=jnp.float32)


def _tc_scale_body(degp_ref, h1_ref, g_ref, dinv_ref):
    deg = degp_ref[0, :, 0:1] + degp_ref[1, :, 0:1] + 1.0
    dinv = lax.rsqrt(deg)
    row = lax.broadcasted_iota(jnp.int32, (N_PAD, 1), 0)
    dinv = jnp.where(row < N, dinv, 0.0)
    dinvb = jnp.broadcast_to(dinv, (N_PAD, D_HID))
    dinv_ref[...] = dinvb
    g_ref[...] = dinvb * h1_ref[...]


def _tc_mid_body(s1p_ref, g_ref, dinv_ref, b1_ref, u_ref):
    s = s1p_ref[0] + s1p_ref[1] + g_ref[...]
    z = jnp.maximum(dinv_ref[...] * s + b1_ref[...], 0.0)
    u_ref[...] = dinv_ref[...] * z


def _tc_final_body(s2p_ref, u_ref, dinv_ref, w2_ref, b2_ref, o_ref):
    y = dinv_ref[...] * (s2p_ref[0] + s2p_ref[1] + u_ref[...])
    logits = jnp.dot(y, w2_ref[...], preferred_element_type=jnp.float32)
    logits = logits[:N] + b2_ref[...]
    m = jnp.max(logits, axis=1, keepdims=True)
    e = jnp.exp(logits - m)
    lse = m + jnp.log(jnp.sum(e, axis=1, keepdims=True))
    o_ref[...] = logits - lse


def _f32(shape):
    return jax.ShapeDtypeStruct(shape, jnp.float32)


def kernel(x, edge_index, W1, b1, W2, b2):
    sc_degree, sc_spmm = _sc_kernels()
    src = edge_index[0]
    dst = edge_index[1]
    pad_e = NW * EPW - E
    src_p = jnp.concatenate(
        [src, jnp.full((pad_e,), N_PAD - 1, jnp.int32)]).reshape(NW, NCHUNK, CHUNK)
    dst_p = jnp.concatenate(
        [dst, jnp.full((pad_e,), N_PAD - 1, jnp.int32)]).reshape(NW, NCHUNK, CHUNK)
    x_pad = jnp.zeros((N_PAD, D_IN), jnp.float32).at[:N].set(x)
    zeros = jnp.zeros((N_PAD, D_HID), jnp.float32)
    ones_rows = jnp.ones((CHUNK, D_HID), jnp.float32)

    h1 = pl.pallas_call(_tc_mm_body, out_shape=_f32((N_PAD, D_HID)))(x_pad, W1)
    degp = sc_degree(dst_p, ones_rows, zeros)
    g1, dinvb = pl.pallas_call(
        _tc_scale_body,
        out_shape=[_f32((N_PAD, D_HID)), _f32((N_PAD, D_HID))])(degp, h1)
    s1p = sc_spmm(g1, src_p, dst_p, zeros)
    u = pl.pallas_call(_tc_mid_body, out_shape=_f32((N_PAD, D_HID)))(
        s1p, g1, dinvb, b1)
    s2p = sc_spmm(u, src_p, dst_p, zeros)
    out = pl.pallas_call(_tc_final_body, out_shape=_f32((N, N_CLS)))(
        s2p, u, dinvb, W2, b2)
    return out


# speedup vs baseline: 23.0305x; 1.1370x over previous
import functools

import jax
import jax.numpy as jnp
from jax import lax
from jax.experimental import pallas as pl
from jax.experimental.pallas import tpu as pltpu
from jax.experimental.pallas import tpu_sc as plsc

N = 10000
E = 160000
D_IN = 256
D_HID = 16
N_CLS = 3

NC = 2
NS = 16
NW = NC * NS
CHUNK = 128
N_PAD = 10240
EPW = 5120
NCHUNK = EPW // CHUNK
ROWS_PER_TILE = N_PAD // NS


@functools.cache
def _sc_kernels():
    mesh = plsc.VectorSubcoreMesh(core_axis_name="c", subcore_axis_name="s")
    partials = jax.ShapeDtypeStruct((NC, N_PAD, D_HID), jnp.float32)
    cparams = pltpu.CompilerParams(use_tc_tiling_on_sc=False)

    @functools.partial(
        pl.kernel,
        out_type=partials,
        mesh=mesh,
        scratch_types=[
            pltpu.VMEM((NCHUNK, CHUNK), jnp.int32),
            pltpu.VMEM((CHUNK, D_HID), jnp.float32),
            pltpu.VMEM_SHARED((N_PAD, D_HID), jnp.float32),
            pltpu.SemaphoreType.DMA,
        ],
        compiler_params=cparams,
    )
    def sc_degree(dst_hbm, ones_hbm, zeros_hbm, out_hbm, dstv, onesv, acc, sem):
        cid = lax.axis_index("c")
        sid = lax.axis_index("s")
        wid = cid * NS + sid
        r0 = sid * ROWS_PER_TILE
        pltpu.sync_copy(zeros_hbm.at[pl.ds(r0, ROWS_PER_TILE)],
                        acc.at[pl.ds(r0, ROWS_PER_TILE)])
        pltpu.sync_copy(ones_hbm, onesv)
        pltpu.sync_copy(dst_hbm.at[wid], dstv)
        plsc.subcore_barrier()

        @pl.loop(0, NCHUNK // 8)
        def _(w):
            @pl.loop(0, 8)
            def _(k):
                pltpu.async_copy(onesv, acc.at[dstv.at[w * 8 + k]], sem,
                                 add=True)

            @pl.loop(0, 8)
            def _(k):
                pltpu.make_async_copy(onesv, acc.at[dstv.at[w * 8 + k]],
                                      sem).wait()

        plsc.subcore_barrier()
        pltpu.sync_copy(acc.at[pl.ds(r0, ROWS_PER_TILE)],
                        out_hbm.at[cid, pl.ds(r0, ROWS_PER_TILE)])

    @functools.partial(
        pl.kernel,
        out_type=partials,
        mesh=mesh,
        scratch_types=[
            pltpu.VMEM((NCHUNK, CHUNK), jnp.int32),
            pltpu.VMEM((NCHUNK, CHUNK), jnp.int32),
            pltpu.VMEM((CHUNK, D_HID), jnp.float32),
            pltpu.VMEM((CHUNK, D_HID), jnp.float32),
            pltpu.VMEM_SHARED((N_PAD, D_HID), jnp.float32),
            pltpu.SemaphoreType.DMA,
            pltpu.SemaphoreType.DMA,
            pltpu.SemaphoreType.DMA,
            pltpu.SemaphoreType.DMA,
        ],
        compiler_params=cparams,
    )
    def sc_spmm(g_hbm, src_hbm, dst_hbm, zeros_hbm, out_hbm,
                srcv, dstv, rows0, rows1, acc, gs0, gs1, ss0, ss1):
        cid = lax.axis_index("c")
        sid = lax.axis_index("s")
        wid = cid * NS + sid
        r0 = sid * ROWS_PER_TILE
        pltpu.sync_copy(zeros_hbm.at[pl.ds(r0, ROWS_PER_TILE)],
                        acc.at[pl.ds(r0, ROWS_PER_TILE)])
        pltpu.sync_copy(src_hbm.at[wid], srcv)
        pltpu.sync_copy(dst_hbm.at[wid], dstv)
        plsc.subcore_barrier()

        pltpu.async_copy(g_hbm.at[srcv.at[0]], rows0, gs0)
        pltpu.async_copy(g_hbm.at[srcv.at[1]], rows1, gs1)

        @pl.loop(0, NCHUNK, step=2)
        def _(j):
            pltpu.make_async_copy(g_hbm.at[srcv.at[j]], rows0, gs0).wait()
            pltpu.async_copy(rows0, acc.at[dstv.at[j]], ss0, add=True)
            pltpu.make_async_copy(g_hbm.at[srcv.at[j + 1]], rows1, gs1).wait()
            pltpu.async_copy(rows1, acc.at[dstv.at[j + 1]], ss1, add=True)
            pltpu.make_async_copy(rows0, acc.at[dstv.at[j]], ss0).wait()

            @pl.when(j + 2 < NCHUNK)
            def _():
                pltpu.async_copy(g_hbm.at[srcv.at[j + 2]], rows0, gs0)

            pltpu.make_async_copy(rows1, acc.at[dstv.at[j + 1]], ss1).wait()

            @pl.when(j + 3 < NCHUNK)
            def _():
                pltpu.async_copy(g_hbm.at[srcv.at[j + 3]], rows1, gs1)

        plsc.subcore_barrier()
        pltpu.sync_copy(acc.at[pl.ds(r0, ROWS_PER_TILE)],
                        out_hbm.at[cid, pl.ds(r0, ROWS_PER_TILE)])

    return sc_degree, sc_spmm


def _tc_mm_body(x_ref, w_ref, o_ref):
    o_ref[...] = jnp.dot(x_ref[...], w_ref[...],
                         preferred_element_type=jnp.float32)


def _tc_scale_body(degp_ref, h1_ref, g_ref, dinv_ref):
    deg = degp_ref[0, :, 0:1] + degp_ref[1, :, 0:1] + 1.0
    dinv = lax.rsqrt(deg)
    row = lax.broadcasted_iota(jnp.int32, (N_PAD, 1), 0)
    dinv = jnp.where(row < N, dinv, 0.0)
    dinvb = jnp.broadcast_to(dinv, (N_PAD, D_HID))
    dinv_ref[...] = dinvb
    g_ref[...] = dinvb * h1_ref[...]


def _tc_mid_body(s1p_ref, g_ref, dinv_ref, b1_ref, u_ref):
    s = s1p_ref[0] + s1p_ref[1] + g_ref[...]
    z = jnp.maximum(dinv_ref[...] * s + b1_ref[...], 0.0)
    u_ref[...] = dinv_ref[...] * z


def _tc_final_body(s2p_ref, u_ref, dinv_ref, w2_ref, b2_ref, o_ref):
    y = dinv_ref[...] * (s2p_ref[0] + s2p_ref[1] + u_ref[...])
    logits = jnp.dot(y, w2_ref[...], preferred_element_type=jnp.float32)
    logits = logits[:N] + b2_ref[...]
    m = jnp.max(logits, axis=1, keepdims=True)
    e = jnp.exp(logits - m)
    lse = m + jnp.log(jnp.sum(e, axis=1, keepdims=True))
    o_ref[...] = logits - lse


def _f32(shape):
    return jax.ShapeDtypeStruct(shape, jnp.float32)


def kernel(x, edge_index, W1, b1, W2, b2):
    sc_degree, sc_spmm = _sc_kernels()
    src = edge_index[0]
    dst = edge_index[1]
    pad_e = NW * EPW - E
    src_p = jnp.concatenate(
        [src, jnp.full((pad_e,), N_PAD - 1, jnp.int32)]).reshape(NW, NCHUNK, CHUNK)
    dst_p = jnp.concatenate(
        [dst, jnp.full((pad_e,), N_PAD - 1, jnp.int32)]).reshape(NW, NCHUNK, CHUNK)
    x_pad = jnp.zeros((N_PAD, D_IN), jnp.float32).at[:N].set(x)
    zeros = jnp.zeros((N_PAD, D_HID), jnp.float32)
    ones_rows = jnp.ones((CHUNK, D_HID), jnp.float32)

    h1 = pl.pallas_call(_tc_mm_body, out_shape=_f32((N_PAD, D_HID)))(x_pad, W1)
    degp = sc_degree(dst_p, ones_rows, zeros)
    g1, dinvb = pl.pallas_call(
        _tc_scale_body,
        out_shape=[_f32((N_PAD, D_HID)), _f32((N_PAD, D_HID))])(degp, h1)
    s1p = sc_spmm(g1, src_p, dst_p, zeros)
    u = pl.pallas_call(_tc_mid_body, out_shape=_f32((N_PAD, D_HID)))(
        s1p, g1, dinvb, b1)
    s2p = sc_spmm(u, src_p, dst_p, zeros)
    out = pl.pallas_call(_tc_final_body, out_shape=_f32((N, N_CLS)))(
        s2p, u, dinvb, W2, b2)
    return out


# speedup vs baseline: 32.4965x; 1.4110x over previous
import functools

import jax
import jax.numpy as jnp
from jax import lax
from jax.experimental import pallas as pl
from jax.experimental.pallas import tpu as pltpu
from jax.experimental.pallas import tpu_sc as plsc

N = 10000
E = 160000
D_IN = 256
D_HID = 16
N_CLS = 3

NC = 2
NS = 16
NW = NC * NS
CHUNK = 128
N_PAD = 10240
EPW = 5120
NCHUNK = EPW // CHUNK
RPT = N_PAD // NS


def _rsqrt16(x):
    i = plsc.bitcast(x, jnp.int32)
    i = jnp.int32(0x5F3759DF) - lax.shift_right_logical(i, 1)
    y = plsc.bitcast(i, jnp.float32)
    for _ in range(3):
        y = y * (1.5 - 0.5 * x * y * y)
    return y


def _edge_loop(src_spmem, srcv, dstv, rows0, rows1, acc, gs0, gs1, ss0, ss1):
    pltpu.async_copy(src_spmem.at[srcv.at[0]], rows0, gs0)
    pltpu.async_copy(src_spmem.at[srcv.at[1]], rows1, gs1)

    @pl.loop(0, NCHUNK, step=2)
    def _(j):
        pltpu.make_async_copy(src_spmem.at[srcv.at[j]], rows0, gs0).wait()
        pltpu.async_copy(rows0, acc.at[dstv.at[j]], ss0, add=True)
        pltpu.make_async_copy(src_spmem.at[srcv.at[j + 1]], rows1, gs1).wait()
        pltpu.async_copy(rows1, acc.at[dstv.at[j + 1]], ss1, add=True)
        pltpu.make_async_copy(rows0, acc.at[dstv.at[j]], ss0).wait()

        @pl.when(j + 2 < NCHUNK)
        def _():
            pltpu.async_copy(src_spmem.at[srcv.at[j + 2]], rows0, gs0)

        pltpu.make_async_copy(rows1, acc.at[dstv.at[j + 1]], ss1).wait()

        @pl.when(j + 3 < NCHUNK)
        def _():
            pltpu.async_copy(src_spmem.at[srcv.at[j + 3]], rows1, gs1)


@functools.cache
def _sc_kernels():
    mesh = plsc.VectorSubcoreMesh(core_axis_name="c", subcore_axis_name="s")
    partials = jax.ShapeDtypeStruct((NC, N_PAD, D_HID), jnp.float32)
    full = jax.ShapeDtypeStruct((N_PAD, D_HID), jnp.float32)
    cparams = pltpu.CompilerParams(use_tc_tiling_on_sc=False,
                                   needs_layout_passes=False)

    @functools.partial(
        pl.kernel,
        out_type=partials,
        mesh=mesh,
        scratch_types=[
            pltpu.VMEM((NCHUNK, CHUNK), jnp.int32),
            pltpu.VMEM((CHUNK, D_HID), jnp.float32),
            pltpu.VMEM_SHARED((N_PAD, D_HID), jnp.float32),
            pltpu.SemaphoreType.DMA,
        ],
        compiler_params=cparams,
    )
    def sc_degree(dst_hbm, ones_hbm, zeros_hbm, out_hbm, dstv, onesv, acc, sem):
        cid = lax.axis_index("c")
        sid = lax.axis_index("s")
        wid = cid * NS + sid
        r0 = sid * RPT
        pltpu.sync_copy(zeros_hbm.at[pl.ds(r0, RPT)], acc.at[pl.ds(r0, RPT)])
        pltpu.sync_copy(ones_hbm, onesv)
        pltpu.sync_copy(dst_hbm.at[wid], dstv)
        plsc.subcore_barrier()

        @pl.loop(0, NCHUNK // 8)
        def _(w):
            @pl.loop(0, 8)
            def _(k):
                pltpu.async_copy(onesv, acc.at[dstv.at[w * 8 + k]], sem,
                                 add=True)

            @pl.loop(0, 8)
            def _(k):
                pltpu.make_async_copy(onesv, acc.at[dstv.at[w * 8 + k]],
                                      sem).wait()

        plsc.subcore_barrier()
        pltpu.sync_copy(acc.at[pl.ds(r0, RPT)], out_hbm.at[cid, pl.ds(r0, RPT)])

    @functools.partial(
        pl.kernel,
        out_type=(partials, full, full),
        mesh=mesh,
        scratch_types=[
            pltpu.VMEM((NCHUNK, CHUNK), jnp.int32),
            pltpu.VMEM((NCHUNK, CHUNK), jnp.int32),
            pltpu.VMEM((CHUNK, D_HID), jnp.float32),
            pltpu.VMEM((CHUNK, D_HID), jnp.float32),
            pltpu.VMEM((RPT, D_HID), jnp.float32),
            pltpu.VMEM((RPT, D_HID), jnp.float32),
            pltpu.VMEM((RPT, D_HID), jnp.float32),
            pltpu.VMEM((RPT, D_HID), jnp.float32),
            pltpu.VMEM_SHARED((N_PAD, D_HID), jnp.float32),
            pltpu.VMEM_SHARED((N_PAD, D_HID), jnp.float32),
            pltpu.SemaphoreType.DMA,
            pltpu.SemaphoreType.DMA,
            pltpu.SemaphoreType.DMA,
            pltpu.SemaphoreType.DMA,
            pltpu.SemaphoreType.DMA,
        ],
        compiler_params=cparams,
    )
    def sc_spmm1(degp_hbm, h1_hbm, src_hbm, dst_hbm, zeros_hbm,
                 s1p_hbm, dinv_hbm, g1_hbm,
                 srcv, dstv, rows0, rows1, d0b, d1b, h1b, g1b,
                 g1s, accs, gs0, gs1, ss0, ss1, ws):
        cid = lax.axis_index("c")
        sid = lax.axis_index("s")
        wid = cid * NS + sid
        r0 = sid * RPT
        pltpu.sync_copy(zeros_hbm.at[pl.ds(r0, RPT)], accs.at[pl.ds(r0, RPT)])
        pltpu.sync_copy(degp_hbm.at[0, pl.ds(r0, RPT)], d0b)
        pltpu.sync_copy(degp_hbm.at[1, pl.ds(r0, RPT)], d1b)
        pltpu.sync_copy(h1_hbm.at[pl.ds(r0, RPT)], h1b)
        pltpu.sync_copy(src_hbm.at[wid], srcv)
        pltpu.sync_copy(dst_hbm.at[wid], dstv)

        @pl.loop(0, RPT)
        def _(i):
            deg = d0b[i, :] + d1b[i, :] + 1.0
            dv = _rsqrt16(deg)
            d0b[i, :] = dv
            g1b[i, :] = dv * h1b[i, :]

        pltpu.sync_copy(g1b, g1s.at[pl.ds(r0, RPT)])
        own = (sid < NS // 2) == (cid == 0)

        @pl.when(own)
        def _():
            pltpu.async_copy(d0b, dinv_hbm.at[pl.ds(r0, RPT)], ws)
            pltpu.async_copy(g1b, g1_hbm.at[pl.ds(r0, RPT)], ws)

        plsc.subcore_barrier()
        _edge_loop(g1s, srcv, dstv, rows0, rows1, accs, gs0, gs1, ss0, ss1)
        plsc.subcore_barrier()
        pltpu.sync_copy(accs.at[pl.ds(r0, RPT)],
                        s1p_hbm.at[cid, pl.ds(r0, RPT)])

        @pl.when(own)
        def _():
            pltpu.make_async_copy(d0b, dinv_hbm.at[pl.ds(r0, RPT)], ws).wait()
            pltpu.make_async_copy(g1b, g1_hbm.at[pl.ds(r0, RPT)], ws).wait()

    @functools.partial(
        pl.kernel,
        out_type=(partials, full),
        mesh=mesh,
        scratch_types=[
            pltpu.VMEM((NCHUNK, CHUNK), jnp.int32),
            pltpu.VMEM((NCHUNK, CHUNK), jnp.int32),
            pltpu.VMEM((CHUNK, D_HID), jnp.float32),
            pltpu.VMEM((CHUNK, D_HID), jnp.float32),
            pltpu.VMEM((RPT, D_HID), jnp.float32),
            pltpu.VMEM((RPT, D_HID), jnp.float32),
            pltpu.VMEM((RPT, D_HID), jnp.float32),
            pltpu.VMEM((RPT, D_HID), jnp.float32),
            pltpu.VMEM((D_HID,), jnp.float32),
            pltpu.VMEM_SHARED((N_PAD, D_HID), jnp.float32),
            pltpu.VMEM_SHARED((N_PAD, D_HID), jnp.float32),
            pltpu.SemaphoreType.DMA,
            pltpu.SemaphoreType.DMA,
            pltpu.SemaphoreType.DMA,
            pltpu.SemaphoreType.DMA,
            pltpu.SemaphoreType.DMA,
        ],
        compiler_params=cparams,
    )
    def sc_spmm2(s1p_hbm, g1_hbm, dinv_hbm, b1_hbm, src_hbm, dst_hbm,
                 zeros_hbm, s2p_hbm, u_hbm,
                 srcv, dstv, rows0, rows1, p0b, p1b, gb, dvb, b1v,
                 us, accs, gs0, gs1, ss0, ss1, ws):
        cid = lax.axis_index("c")
        sid = lax.axis_index("s")
        wid = cid * NS + sid
        r0 = sid * RPT
        pltpu.sync_copy(zeros_hbm.at[pl.ds(r0, RPT)], accs.at[pl.ds(r0, RPT)])
        pltpu.sync_copy(s1p_hbm.at[0, pl.ds(r0, RPT)], p0b)
        pltpu.sync_copy(s1p_hbm.at[1, pl.ds(r0, RPT)], p1b)
        pltpu.sync_copy(g1_hbm.at[pl.ds(r0, RPT)], gb)
        pltpu.sync_copy(dinv_hbm.at[pl.ds(r0, RPT)], dvb)
        pltpu.sync_copy(b1_hbm, b1v)
        pltpu.sync_copy(src_hbm.at[wid], srcv)
        pltpu.sync_copy(dst_hbm.at[wid], dstv)
        b1r = b1v[...]

        @pl.loop(0, RPT)
        def _(i):
            dv = dvb[i, :]
            z = jnp.maximum(dv * (gb[i, :] + p0b[i, :] + p1b[i, :]) + b1r, 0.0)
            gb[i, :] = dv * z

        pltpu.sync_copy(gb, us.at[pl.ds(r0, RPT)])
        own = (sid < NS // 2) == (cid == 0)

        @pl.when(own)
        def _():
            pltpu.async_copy(gb, u_hbm.at[pl.ds(r0, RPT)], ws)

        plsc.subcore_barrier()
        _edge_loop(us, srcv, dstv, rows0, rows1, accs, gs0, gs1, ss0, ss1)
        plsc.subcore_barrier()
        pltpu.sync_copy(accs.at[pl.ds(r0, RPT)],
                        s2p_hbm.at[cid, pl.ds(r0, RPT)])

        @pl.when(own)
        def _():
            pltpu.make_async_copy(gb, u_hbm.at[pl.ds(r0, RPT)], ws).wait()

    return sc_degree, sc_spmm1, sc_spmm2


def _tc_mm_body(x_ref, w_ref, o_ref):
    o_ref[...] = jnp.dot(x_ref[...], w_ref[...],
                         preferred_element_type=jnp.float32)


def _tc_final_body(s2p_ref, u_ref, dinv_ref, w2_ref, b2_ref, o_ref):
    y = dinv_ref[...] * (s2p_ref[0] + s2p_ref[1] + u_ref[...])
    logits = jnp.dot(y, w2_ref[...], preferred_element_type=jnp.float32)
    logits = logits[:N] + b2_ref[...]
    m = jnp.max(logits, axis=1, keepdims=True)
    e = jnp.exp(logits - m)
    lse = m + jnp.log(jnp.sum(e, axis=1, keepdims=True))
    o_ref[...] = logits - lse


def _f32(shape):
    return jax.ShapeDtypeStruct(shape, jnp.float32)


def kernel(x, edge_index, W1, b1, W2, b2):
    sc_degree, sc_spmm1, sc_spmm2 = _sc_kernels()
    src = edge_index[0]
    dst = edge_index[1]
    pad_e = NW * EPW - E
    src_p = jnp.concatenate(
        [src, jnp.full((pad_e,), N_PAD - 1, jnp.int32)]).reshape(NW, NCHUNK, CHUNK)
    dst_p = jnp.concatenate(
        [dst, jnp.full((pad_e,), N_PAD - 1, jnp.int32)]).reshape(NW, NCHUNK, CHUNK)
    x_pad = jnp.zeros((N_PAD, D_IN), jnp.float32).at[:N].set(x)
    zeros = jnp.zeros((N_PAD, D_HID), jnp.float32)
    ones_rows = jnp.ones((CHUNK, D_HID), jnp.float32)

    h1 = pl.pallas_call(_tc_mm_body, out_shape=_f32((N_PAD, D_HID)))(x_pad, W1)
    degp = sc_degree(dst_p, ones_rows, zeros)
    s1p, dinvb, g1 = sc_spmm1(degp, h1, src_p, dst_p, zeros)
    s2p, u = sc_spmm2(s1p, g1, dinvb, b1, src_p, dst_p, zeros)
    out = pl.pallas_call(_tc_final_body, out_shape=_f32((N, N_CLS)))(
        s2p, u, dinvb, W2, b2)
    return out


# speedup vs baseline: 37.6965x; 1.1600x over previous
import functools

import jax
import jax.numpy as jnp
from jax import lax
from jax.experimental import pallas as pl
from jax.experimental.pallas import tpu as pltpu
from jax.experimental.pallas import tpu_sc as plsc

N = 10000
E = 160000
D_IN = 256
D_HID = 16
N_CLS = 3

NC = 2
NS = 16
NW = NC * NS
CHUNK = 128
N_PAD = 10240
EPW = E // NW
NCHUNK = 40
RPT = N_PAD // NS
DUMP = N_PAD - 1
TAIL = EPW - (NCHUNK - 1) * CHUNK


def _rsqrt16(x):
    i = plsc.bitcast(x, jnp.int32)
    i = jnp.int32(0x5F3759DF) - lax.shift_right_logical(i, 1)
    y = plsc.bitcast(i, jnp.float32)
    for _ in range(3):
        y = y * (1.5 - 0.5 * x * y * y)
    return y


def _load_slab(edge_hbm, row, wid, buf):
    pltpu.sync_copy(edge_hbm.at[row, pl.ds(wid * EPW, EPW)],
                    buf.at[pl.ds(0, EPW)])
    lanes = lax.iota(jnp.int32, 16)
    first = (NCHUNK - 1) * CHUNK + (TAIL // 16) * 16
    real = buf[pl.ds(first, 16)]
    buf[pl.ds(first, 16)] = jnp.where(lanes < (EPW - first),
                                      real, jnp.int32(DUMP))
    for k in range(first + 16, NCHUNK * CHUNK, 16):
        buf[pl.ds(k, 16)] = jnp.full((16,), DUMP, jnp.int32)


def _build_2d(buf, idx2d):
    @pl.loop(0, NCHUNK)
    def _(j):
        for t in range(CHUNK // 16):
            idx2d[j, pl.ds(t * 16, 16)] = buf[pl.ds(j * CHUNK + t * 16, 16)]


def _edge_loop(src_spmem, sall, dst2d, rows0, rows1, acc, gs0, gs1, ss0, ss1):
    pltpu.async_copy(src_spmem.at[sall.at[pl.ds(0, CHUNK)]], rows0, gs0)
    pltpu.async_copy(src_spmem.at[sall.at[pl.ds(CHUNK, CHUNK)]], rows1, gs1)

    @pl.loop(0, NCHUNK, step=2)
    def _(j):
        s0 = src_spmem.at[sall.at[pl.ds(j * CHUNK, CHUNK)]]
        s1 = src_spmem.at[sall.at[pl.ds((j + 1) * CHUNK, CHUNK)]]
        pltpu.make_async_copy(s0, rows0, gs0).wait()
        pltpu.async_copy(rows0, acc.at[dst2d.at[j]], ss0, add=True)
        pltpu.make_async_copy(s1, rows1, gs1).wait()
        pltpu.async_copy(rows1, acc.at[dst2d.at[j + 1]], ss1, add=True)
        pltpu.make_async_copy(rows0, acc.at[dst2d.at[j]], ss0).wait()

        @pl.when(j + 2 < NCHUNK)
        def _():
            nxt = src_spmem.at[sall.at[pl.ds((j + 2) * CHUNK, CHUNK)]]
            pltpu.async_copy(nxt, rows0, gs0)

        pltpu.make_async_copy(rows1, acc.at[dst2d.at[j + 1]], ss1).wait()

        @pl.when(j + 3 < NCHUNK)
        def _():
            nxt = src_spmem.at[sall.at[pl.ds((j + 3) * CHUNK, CHUNK)]]
            pltpu.async_copy(nxt, rows1, gs1)


@functools.cache
def _sc_kernels():
    mesh = plsc.VectorSubcoreMesh(core_axis_name="c", subcore_axis_name="s")
    partials = jax.ShapeDtypeStruct((NC, N_PAD, D_HID), jnp.float32)
    full = jax.ShapeDtypeStruct((N_PAD, D_HID), jnp.float32)
    cparams = pltpu.CompilerParams(use_tc_tiling_on_sc=False,
                                   needs_layout_passes=False)

    @functools.partial(
        pl.kernel,
        out_type=partials,
        mesh=mesh,
        scratch_types=[
            pltpu.VMEM((NCHUNK * CHUNK,), jnp.int32),
            pltpu.VMEM((NCHUNK, CHUNK), jnp.int32),
            pltpu.VMEM((CHUNK, D_HID), jnp.float32),
            pltpu.VMEM_SHARED((N_PAD, D_HID), jnp.float32),
            pltpu.SemaphoreType.DMA,
        ],
        compiler_params=cparams,
    )
    def sc_degree(edge_hbm, ones_hbm, zeros_hbm, out_hbm,
                  dall, dst2d, onesv, acc, sem):
        cid = lax.axis_index("c")
        sid = lax.axis_index("s")
        wid = cid * NS + sid
        r0 = sid * RPT
        pltpu.sync_copy(zeros_hbm.at[pl.ds(r0, RPT)], acc.at[pl.ds(r0, RPT)])
        pltpu.sync_copy(ones_hbm, onesv)
        _load_slab(edge_hbm, 1, wid, dall)
        _build_2d(dall, dst2d)
        plsc.subcore_barrier()

        @pl.loop(0, NCHUNK // 8)
        def _(w):
            @pl.loop(0, 8)
            def _(k):
                pltpu.async_copy(onesv, acc.at[dst2d.at[w * 8 + k]], sem,
                                 add=True)

            @pl.loop(0, 8)
            def _(k):
                pltpu.make_async_copy(onesv, acc.at[dst2d.at[w * 8 + k]],
                                      sem).wait()

        plsc.subcore_barrier()
        pltpu.sync_copy(acc.at[pl.ds(r0, RPT)], out_hbm.at[cid, pl.ds(r0, RPT)])

    @functools.partial(
        pl.kernel,
        out_type=(partials, full),
        mesh=mesh,
        scratch_types=[
            pltpu.VMEM((NCHUNK * CHUNK,), jnp.int32),
            pltpu.VMEM((NCHUNK * CHUNK,), jnp.int32),
            pltpu.VMEM((NCHUNK, CHUNK), jnp.int32),
            pltpu.VMEM((CHUNK, D_HID), jnp.float32),
            pltpu.VMEM((CHUNK, D_HID), jnp.float32),
            pltpu.VMEM((RPT, D_HID), jnp.float32),
            pltpu.VMEM((RPT, D_HID), jnp.float32),
            pltpu.VMEM((RPT, D_HID), jnp.float32),
            pltpu.VMEM((RPT, D_HID), jnp.float32),
            pltpu.VMEM_SHARED((N_PAD, D_HID), jnp.float32),
            pltpu.VMEM_SHARED((N_PAD, D_HID), jnp.float32),
            pltpu.SemaphoreType.DMA,
            pltpu.SemaphoreType.DMA,
            pltpu.SemaphoreType.DMA,
            pltpu.SemaphoreType.DMA,
            pltpu.SemaphoreType.DMA,
        ],
        compiler_params=cparams,
    )
    def sc_spmm1(degp_hbm, h1_hbm, edge_hbm, zeros_hbm,
                 s1p_hbm, dinv_hbm,
                 sall, dall, dst2d, rows0, rows1, d0b, d1b, h1b, g1b,
                 g1s, accs, gs0, gs1, ss0, ss1, ws):
        cid = lax.axis_index("c")
        sid = lax.axis_index("s")
        wid = cid * NS + sid
        r0 = sid * RPT
        pltpu.sync_copy(degp_hbm.at[0, pl.ds(r0, RPT)], d0b)
        pltpu.sync_copy(degp_hbm.at[1, pl.ds(r0, RPT)], d1b)
        pltpu.sync_copy(h1_hbm.at[pl.ds(r0, RPT)], h1b)
        _load_slab(edge_hbm, 0, wid, sall)
        _load_slab(edge_hbm, 1, wid, dall)
        _build_2d(dall, dst2d)

        @pl.loop(0, RPT)
        def _(i):
            deg = d0b[i, :] + d1b[i, :] + 1.0
            dv = _rsqrt16(deg)
            d0b[i, :] = dv
            g1b[i, :] = dv * h1b[i, :]

        pltpu.sync_copy(g1b, g1s.at[pl.ds(r0, RPT)])

        @pl.when(cid == 0)
        def _():
            pltpu.sync_copy(g1b, accs.at[pl.ds(r0, RPT)])

        @pl.when(cid != 0)
        def _():
            pltpu.sync_copy(zeros_hbm.at[pl.ds(r0, RPT)],
                            accs.at[pl.ds(r0, RPT)])

        own = (sid < NS // 2) == (cid == 0)

        @pl.when(own)
        def _():
            pltpu.async_copy(d0b, dinv_hbm.at[pl.ds(r0, RPT)], ws)

        plsc.subcore_barrier()
        _edge_loop(g1s, sall, dst2d, rows0, rows1, accs, gs0, gs1, ss0, ss1)
        plsc.subcore_barrier()
        pltpu.sync_copy(accs.at[pl.ds(r0, RPT)],
                        s1p_hbm.at[cid, pl.ds(r0, RPT)])

        @pl.when(own)
        def _():
            pltpu.make_async_copy(d0b, dinv_hbm.at[pl.ds(r0, RPT)], ws).wait()

    @functools.partial(
        pl.kernel,
        out_type=partials,
        mesh=mesh,
        scratch_types=[
            pltpu.VMEM((NCHUNK * CHUNK,), jnp.int32),
            pltpu.VMEM((NCHUNK * CHUNK,), jnp.int32),
            pltpu.VMEM((NCHUNK, CHUNK), jnp.int32),
            pltpu.VMEM((CHUNK, D_HID), jnp.float32),
            pltpu.VMEM((CHUNK, D_HID), jnp.float32),
            pltpu.VMEM((RPT, D_HID), jnp.float32),
            pltpu.VMEM((RPT, D_HID), jnp.float32),
            pltpu.VMEM((RPT, D_HID), jnp.float32),
            pltpu.VMEM((D_HID,), jnp.float32),
            pltpu.VMEM_SHARED((N_PAD, D_HID), jnp.float32),
            pltpu.VMEM_SHARED((N_PAD, D_HID), jnp.float32),
            pltpu.SemaphoreType.DMA,
            pltpu.SemaphoreType.DMA,
            pltpu.SemaphoreType.DMA,
            pltpu.SemaphoreType.DMA,
        ],
        compiler_params=cparams,
    )
    def sc_spmm2(s1p_hbm, dinv_hbm, b1_hbm, edge_hbm, zeros_hbm, s2p_hbm,
                 sall, dall, dst2d, rows0, rows1, p0b, p1b, dvb, b1v,
                 us, accs, gs0, gs1, ss0, ss1):
        cid = lax.axis_index("c")
        sid = lax.axis_index("s")
        wid = cid * NS + sid
        r0 = sid * RPT
        pltpu.sync_copy(s1p_hbm.at[0, pl.ds(r0, RPT)], p0b)
        pltpu.sync_copy(s1p_hbm.at[1, pl.ds(r0, RPT)], p1b)
        pltpu.sync_copy(dinv_hbm.at[pl.ds(r0, RPT)], dvb)
        pltpu.sync_copy(b1_hbm, b1v)
        _load_slab(edge_hbm, 0, wid, sall)
        _load_slab(edge_hbm, 1, wid, dall)
        _build_2d(dall, dst2d)
        b1r = b1v[...]

        @pl.loop(0, RPT)
        def _(i):
            dv = dvb[i, :]
            z = jnp.maximum(dv * (p0b[i, :] + p1b[i, :]) + b1r, 0.0)
            p0b[i, :] = dv * z

        pltpu.sync_copy(p0b, us.at[pl.ds(r0, RPT)])

        @pl.when(cid == 0)
        def _():
            pltpu.sync_copy(p0b, accs.at[pl.ds(r0, RPT)])

        @pl.when(cid != 0)
        def _():
            pltpu.sync_copy(zeros_hbm.at[pl.ds(r0, RPT)],
                            accs.at[pl.ds(r0, RPT)])

        plsc.subcore_barrier()
        _edge_loop(us, sall, dst2d, rows0, rows1, accs, gs0, gs1, ss0, ss1)
        plsc.subcore_barrier()

        pltpu.sync_copy(accs.at[pl.ds(r0, RPT)], p1b)

        @pl.loop(0, RPT)
        def _(i):
            p1b[i, :] = p1b[i, :] * dvb[i, :]

        pltpu.sync_copy(p1b, s2p_hbm.at[cid, pl.ds(r0, RPT)])

    return sc_degree, sc_spmm1, sc_spmm2


def _tc_mm_body(x_ref, w_ref, o_ref):
    h = jnp.dot(x_ref[...], w_ref[...], preferred_element_type=jnp.float32)
    o_ref[...] = jnp.pad(h, ((0, N_PAD - N), (0, 0)))


def _tc_final_body(s2p_ref, w2_ref, b2_ref, o_ref):
    y = s2p_ref[0] + s2p_ref[1]
    logits = jnp.dot(y, w2_ref[...], preferred_element_type=jnp.float32)
    logits = logits[:N] + b2_ref[...]
    m = jnp.max(logits, axis=1, keepdims=True)
    e = jnp.exp(logits - m)
    lse = m + jnp.log(jnp.sum(e, axis=1, keepdims=True))
    o_ref[...] = logits - lse


def _f32(shape):
    return jax.ShapeDtypeStruct(shape, jnp.float32)


def kernel(x, edge_index, W1, b1, W2, b2):
    sc_degree, sc_spmm1, sc_spmm2 = _sc_kernels()
    zeros = jnp.zeros((N_PAD, D_HID), jnp.float32)
    ones_rows = jnp.ones((CHUNK, D_HID), jnp.float32)

    h1 = pl.pallas_call(_tc_mm_body, out_shape=_f32((N_PAD, D_HID)))(x, W1)
    degp = sc_degree(edge_index, ones_rows, zeros)
    s1p, dinvb = sc_spmm1(degp, h1, edge_index, zeros)
    s2p = sc_spmm2(s1p, dinvb, b1, edge_index, zeros)
    out = pl.pallas_call(_tc_final_body, out_shape=_f32((N, N_CLS)))(
        s2p, W2, b2)
    return out


# speedup vs baseline: 39.7578x; 1.0547x over previous
import functools

import jax
import jax.numpy as jnp
from jax import lax
from jax.experimental import pallas as pl
from jax.experimental.pallas import tpu as pltpu
from jax.experimental.pallas import tpu_sc as plsc

N = 10000
E = 160000
D_IN = 256
D_HID = 16
N_CLS = 3

NC = 2
NS = 16
NW = NC * NS
CHUNK = 128
N_PAD = 10240
EPW = E // NW
NCHUNK = 40
RPT = N_PAD // NS
DUMP = N_PAD - 1
TAIL = EPW - (NCHUNK - 1) * CHUNK


_GATHER_DNUMS = lax.GatherDimensionNumbers(
    offset_dims=(), collapsed_slice_dims=(0,), start_index_map=(0,))


def _bcast_lane(v, l):
    idx = jnp.full((16, 1), l, jnp.int32)
    return lax.gather(v, idx, _GATHER_DNUMS, (1,),
                      mode=lax.GatherScatterMode.PROMISE_IN_BOUNDS)


def _rsqrt16(x):
    i = plsc.bitcast(x, jnp.int32)
    i = jnp.int32(0x5F3759DF) - lax.shift_right_logical(i, 1)
    y = plsc.bitcast(i, jnp.float32)
    for _ in range(3):
        y = y * (1.5 - 0.5 * x * y * y)
    return y


def _load_slab(edge_hbm, row, wid, buf):
    pltpu.sync_copy(edge_hbm.at[row, pl.ds(wid * EPW, EPW)],
                    buf.at[pl.ds(0, EPW)])
    lanes = lax.iota(jnp.int32, 16)
    first = (NCHUNK - 1) * CHUNK + (TAIL // 16) * 16
    real = buf[pl.ds(first, 16)]
    buf[pl.ds(first, 16)] = jnp.where(lanes < (EPW - first),
                                      real, jnp.int32(DUMP))
    for k in range(first + 16, NCHUNK * CHUNK, 16):
        buf[pl.ds(k, 16)] = jnp.full((16,), DUMP, jnp.int32)


def _build_2d(buf, idx2d):
    @pl.loop(0, NCHUNK)
    def _(j):
        for t in range(CHUNK // 16):
            idx2d[j, pl.ds(t * 16, 16)] = buf[pl.ds(j * CHUNK + t * 16, 16)]


def _edge_loop(src_spmem, sall, dst2d, rows0, rows1, acc, gs0, gs1, ss0, ss1):
    pltpu.async_copy(src_spmem.at[sall.at[pl.ds(0, CHUNK)]], rows0, gs0)
    pltpu.async_copy(src_spmem.at[sall.at[pl.ds(CHUNK, CHUNK)]], rows1, gs1)

    @pl.loop(0, NCHUNK, step=2)
    def _(j):
        s0 = src_spmem.at[sall.at[pl.ds(j * CHUNK, CHUNK)]]
        s1 = src_spmem.at[sall.at[pl.ds((j + 1) * CHUNK, CHUNK)]]
        pltpu.make_async_copy(s0, rows0, gs0).wait()
        pltpu.async_copy(rows0, acc.at[dst2d.at[j]], ss0, add=True)
        pltpu.make_async_copy(s1, rows1, gs1).wait()
        pltpu.async_copy(rows1, acc.at[dst2d.at[j + 1]], ss1, add=True)
        pltpu.make_async_copy(rows0, acc.at[dst2d.at[j]], ss0).wait()

        @pl.when(j + 2 < NCHUNK)
        def _():
            nxt = src_spmem.at[sall.at[pl.ds((j + 2) * CHUNK, CHUNK)]]
            pltpu.async_copy(nxt, rows0, gs0)

        pltpu.make_async_copy(rows1, acc.at[dst2d.at[j + 1]], ss1).wait()

        @pl.when(j + 3 < NCHUNK)
        def _():
            nxt = src_spmem.at[sall.at[pl.ds((j + 3) * CHUNK, CHUNK)]]
            pltpu.async_copy(nxt, rows1, gs1)


@functools.cache
def _sc_kernels():
    mesh = plsc.VectorSubcoreMesh(core_axis_name="c", subcore_axis_name="s")
    partials = jax.ShapeDtypeStruct((NC, N_PAD, D_HID), jnp.float32)
    full = jax.ShapeDtypeStruct((N_PAD, D_HID), jnp.float32)
    cparams = pltpu.CompilerParams(use_tc_tiling_on_sc=False,
                                   needs_layout_passes=False)

    @functools.partial(
        pl.kernel,
        out_type=(partials, full),
        mesh=mesh,
        scratch_types=[
            pltpu.VMEM((NCHUNK * CHUNK,), jnp.int32),
            pltpu.VMEM((NCHUNK * CHUNK,), jnp.int32),
            pltpu.VMEM((NCHUNK, CHUNK), jnp.int32),
            pltpu.VMEM((E // NS,), jnp.int32),
            pltpu.VMEM((N_PAD,), jnp.float32),
            pltpu.VMEM((NS, RPT), jnp.float32),
            pltpu.VMEM((CHUNK, D_HID), jnp.float32),
            pltpu.VMEM((CHUNK, D_HID), jnp.float32),
            pltpu.VMEM((RPT, D_HID), jnp.float32),
            pltpu.VMEM((RPT, D_HID), jnp.float32),
            pltpu.VMEM((RPT, D_HID), jnp.float32),
            pltpu.VMEM_SHARED((NS, N_PAD), jnp.float32),
            pltpu.VMEM_SHARED((N_PAD, D_HID), jnp.float32),
            pltpu.VMEM_SHARED((N_PAD, D_HID), jnp.float32),
            pltpu.SemaphoreType.DMA,
            pltpu.SemaphoreType.DMA,
            pltpu.SemaphoreType.DMA,
            pltpu.SemaphoreType.DMA,
            pltpu.SemaphoreType.DMA,
        ],
        compiler_params=cparams,
    )
    def sc_spmm1(h1_hbm, edge_hbm, zeros_hbm,
                 s1p_hbm, dinv_hbm,
                 sall, dall, dst2d, dalla, degacc, dred, rows0, rows1,
                 h1b, g1b, dvb,
                 degsh, g1s, accs, gs0, gs1, ss0, ss1, ws):
        cid = lax.axis_index("c")
        sid = lax.axis_index("s")
        wid = cid * NS + sid
        r0 = sid * RPT
        dpt = E // NS
        ones16 = jnp.ones((D_HID,), jnp.float32)
        zero16 = jnp.zeros((D_HID,), jnp.float32)
        pltpu.sync_copy(h1_hbm.at[pl.ds(r0, RPT)], h1b)
        pltpu.sync_copy(edge_hbm.at[1, pl.ds(sid * dpt, dpt)], dalla)
        _load_slab(edge_hbm, 0, wid, sall)
        _load_slab(edge_hbm, 1, wid, dall)
        _build_2d(dall, dst2d)

        @pl.loop(0, N_PAD // 16)
        def _(i):
            degacc[pl.ds(i * 16, 16)] = zero16

        @pl.loop(0, dpt // 16)
        def _(k):
            idx = dalla[pl.ds(k * 16, 16)]
            plsc.addupdate_scatter(degacc, [idx], ones16)

        pltpu.sync_copy(degacc, degsh.at[sid])
        plsc.subcore_barrier()
        pltpu.sync_copy(degsh.at[pl.ds(0, NS), pl.ds(r0, RPT)], dred)

        @pl.loop(0, RPT // 16)
        def _(v):
            dsum = dred[0, pl.ds(v * 16, 16)]
            for r in range(1, NS):
                dsum = dsum + dred[r, pl.ds(v * 16, 16)]
            dv = _rsqrt16(dsum + 1.0)
            for l in range(16):
                bc = _bcast_lane(dv, l)
                dvb[v * 16 + l, :] = bc
                g1b[v * 16 + l, :] = bc * h1b[v * 16 + l, :]

        pltpu.sync_copy(g1b, g1s.at[pl.ds(r0, RPT)])

        @pl.when(cid == 0)
        def _():
            pltpu.sync_copy(g1b, accs.at[pl.ds(r0, RPT)])

        @pl.when(cid != 0)
        def _():
            pltpu.sync_copy(zeros_hbm.at[pl.ds(r0, RPT)],
                            accs.at[pl.ds(r0, RPT)])

        own = (sid < NS // 2) == (cid == 0)

        @pl.when(own)
        def _():
            pltpu.async_copy(dvb, dinv_hbm.at[pl.ds(r0, RPT)], ws)

        plsc.subcore_barrier()
        _edge_loop(g1s, sall, dst2d, rows0, rows1, accs, gs0, gs1, ss0, ss1)
        plsc.subcore_barrier()
        pltpu.sync_copy(accs.at[pl.ds(r0, RPT)],
                        s1p_hbm.at[cid, pl.ds(r0, RPT)])

        @pl.when(own)
        def _():
            pltpu.make_async_copy(dvb, dinv_hbm.at[pl.ds(r0, RPT)], ws).wait()

    @functools.partial(
        pl.kernel,
        out_type=partials,
        mesh=mesh,
        scratch_types=[
            pltpu.VMEM((NCHUNK * CHUNK,), jnp.int32),
            pltpu.VMEM((NCHUNK * CHUNK,), jnp.int32),
            pltpu.VMEM((NCHUNK, CHUNK), jnp.int32),
            pltpu.VMEM((CHUNK, D_HID), jnp.float32),
            pltpu.VMEM((CHUNK, D_HID), jnp.float32),
            pltpu.VMEM((RPT, D_HID), jnp.float32),
            pltpu.VMEM((RPT, D_HID), jnp.float32),
            pltpu.VMEM((RPT, D_HID), jnp.float32),
            pltpu.VMEM((D_HID,), jnp.float32),
            pltpu.VMEM_SHARED((N_PAD, D_HID), jnp.float32),
            pltpu.VMEM_SHARED((N_PAD, D_HID), jnp.float32),
            pltpu.SemaphoreType.DMA,
            pltpu.SemaphoreType.DMA,
            pltpu.SemaphoreType.DMA,
            pltpu.SemaphoreType.DMA,
        ],
        compiler_params=cparams,
    )
    def sc_spmm2(s1p_hbm, dinv_hbm, b1_hbm, edge_hbm, zeros_hbm, s2p_hbm,
                 sall, dall, dst2d, rows0, rows1, p0b, p1b, dvb, b1v,
                 us, accs, gs0, gs1, ss0, ss1):
        cid = lax.axis_index("c")
        sid = lax.axis_index("s")
        wid = cid * NS + sid
        r0 = sid * RPT
        pltpu.sync_copy(s1p_hbm.at[0, pl.ds(r0, RPT)], p0b)
        pltpu.sync_copy(s1p_hbm.at[1, pl.ds(r0, RPT)], p1b)
        pltpu.sync_copy(dinv_hbm.at[pl.ds(r0, RPT)], dvb)
        pltpu.sync_copy(b1_hbm, b1v)
        _load_slab(edge_hbm, 0, wid, sall)
        _load_slab(edge_hbm, 1, wid, dall)
        _build_2d(dall, dst2d)
        b1r = b1v[...]

        @pl.loop(0, RPT)
        def _(i):
            dv = dvb[i, :]
            z = jnp.maximum(dv * (p0b[i, :] + p1b[i, :]) + b1r, 0.0)
            p0b[i, :] = dv * z

        pltpu.sync_copy(p0b, us.at[pl.ds(r0, RPT)])

        @pl.when(cid == 0)
        def _():
            pltpu.sync_copy(p0b, accs.at[pl.ds(r0, RPT)])

        @pl.when(cid != 0)
        def _():
            pltpu.sync_copy(zeros_hbm.at[pl.ds(r0, RPT)],
                            accs.at[pl.ds(r0, RPT)])

        plsc.subcore_barrier()
        _edge_loop(us, sall, dst2d, rows0, rows1, accs, gs0, gs1, ss0, ss1)
        plsc.subcore_barrier()

        pltpu.sync_copy(accs.at[pl.ds(r0, RPT)], p1b)

        @pl.loop(0, RPT)
        def _(i):
            p1b[i, :] = p1b[i, :] * dvb[i, :]

        pltpu.sync_copy(p1b, s2p_hbm.at[cid, pl.ds(r0, RPT)])

    return sc_spmm1, sc_spmm2


def _tc_mm_body(x_ref, w_ref, o_ref):
    h = jnp.dot(x_ref[...], w_ref[...], preferred_element_type=jnp.float32)
    o_ref[...] = jnp.pad(h, ((0, N_PAD - N), (0, 0)))


def _tc_final_body(s2p_ref, w2_ref, b2_ref, o_ref):
    y = s2p_ref[0] + s2p_ref[1]
    w2p = jnp.pad(w2_ref[...], ((0, 0), (0, D_HID - N_CLS)))
    b2p = jnp.pad(b2_ref[...], (0, D_HID - N_CLS), constant_values=-1e30)
    logits = jnp.dot(y, w2p, preferred_element_type=jnp.float32) + b2p
    m = jnp.max(logits, axis=1, keepdims=True)
    e = jnp.exp(logits - m)
    lse = m + jnp.log(jnp.sum(e, axis=1, keepdims=True))
    o_ref[...] = (logits - lse)[:N, :N_CLS]


def _f32(shape):
    return jax.ShapeDtypeStruct(shape, jnp.float32)


def kernel(x, edge_index, W1, b1, W2, b2):
    sc_spmm1, sc_spmm2 = _sc_kernels()
    zeros = jnp.zeros((N_PAD, D_HID), jnp.float32)

    h1 = pl.pallas_call(_tc_mm_body, out_shape=_f32((N_PAD, D_HID)))(x, W1)
    s1p, dinvb = sc_spmm1(h1, edge_index, zeros)
    s2p = sc_spmm2(s1p, dinvb, b1, edge_index, zeros)
    out = pl.pallas_call(_tc_final_body, out_shape=_f32((N, N_CLS)))(
        s2p, W2, b2)
    return out


# speedup vs baseline: 40.5860x; 1.0208x over previous
import functools

import jax
import jax.numpy as jnp
from jax import lax
from jax.experimental import pallas as pl
from jax.experimental.pallas import tpu as pltpu
from jax.experimental.pallas import tpu_sc as plsc

N = 10000
E = 160000
D_IN = 256
D_HID = 16
N_CLS = 3

NC = 2
NS = 16
NW = NC * NS
CHUNK = 128
N_PAD = 10240
EPW = E // NW
NCHUNK = 40
RPT = N_PAD // NS
DUMP = N_PAD - 1
TAIL = EPW - (NCHUNK - 1) * CHUNK


_GATHER_DNUMS = lax.GatherDimensionNumbers(
    offset_dims=(), collapsed_slice_dims=(0,), start_index_map=(0,))


def _bcast_lane(v, l):
    idx = jnp.full((16, 1), l, jnp.int32)
    return lax.gather(v, idx, _GATHER_DNUMS, (1,),
                      mode=lax.GatherScatterMode.PROMISE_IN_BOUNDS)


def _rsqrt16(x):
    i = plsc.bitcast(x, jnp.int32)
    i = jnp.int32(0x5F3759DF) - lax.shift_right_logical(i, 1)
    y = plsc.bitcast(i, jnp.float32)
    for _ in range(3):
        y = y * (1.5 - 0.5 * x * y * y)
    return y


def _load_slab(edge_hbm, row, wid, buf):
    pltpu.sync_copy(edge_hbm.at[row, pl.ds(wid * EPW, EPW)],
                    buf.at[pl.ds(0, EPW)])
    lanes = lax.iota(jnp.int32, 16)
    first = (NCHUNK - 1) * CHUNK + (TAIL // 16) * 16
    real = buf[pl.ds(first, 16)]
    buf[pl.ds(first, 16)] = jnp.where(lanes < (EPW - first),
                                      real, jnp.int32(DUMP))
    for k in range(first + 16, NCHUNK * CHUNK, 16):
        buf[pl.ds(k, 16)] = jnp.full((16,), DUMP, jnp.int32)


def _build_2d(buf, idx2d):
    @pl.loop(0, NCHUNK)
    def _(j):
        for t in range(CHUNK // 16):
            idx2d[j, pl.ds(t * 16, 16)] = buf[pl.ds(j * CHUNK + t * 16, 16)]


NBUF = 4


def _edge_loop(src_spmem, sall, dst2d, rows, gs, ss, acc):
    def gref(j):
        return src_spmem.at[sall.at[pl.ds(j * CHUNK, CHUNK)]]

    for b in range(NBUF):
        pltpu.async_copy(gref(b), rows[b], gs[b])

    @pl.loop(0, NCHUNK, step=NBUF)
    def _(j):
        for b in range(NBUF):
            pltpu.make_async_copy(gref(j + b), rows[b], gs[b]).wait()
            pltpu.async_copy(rows[b], acc.at[dst2d.at[j + b]], ss[b], add=True)
        for b in range(NBUF):
            pltpu.make_async_copy(rows[b], acc.at[dst2d.at[j + b]],
                                  ss[b]).wait()

            @pl.when(j + NBUF + b < NCHUNK)
            def _():
                pltpu.async_copy(gref(j + NBUF + b), rows[b], gs[b])


@functools.cache
def _sc_kernels():
    mesh = plsc.VectorSubcoreMesh(core_axis_name="c", subcore_axis_name="s")
    partials = jax.ShapeDtypeStruct((NC, N_PAD, D_HID), jnp.float32)
    full = jax.ShapeDtypeStruct((N_PAD, D_HID), jnp.float32)
    cparams = pltpu.CompilerParams(use_tc_tiling_on_sc=False,
                                   needs_layout_passes=False)

    @functools.partial(
        pl.kernel,
        out_type=(partials, full),
        mesh=mesh,
        scratch_types=[
            pltpu.VMEM((NCHUNK * CHUNK,), jnp.int32),
            pltpu.VMEM((NCHUNK * CHUNK,), jnp.int32),
            pltpu.VMEM((NCHUNK, CHUNK), jnp.int32),
            pltpu.VMEM((E // NS,), jnp.int32),
            pltpu.VMEM((N_PAD,), jnp.float32),
            pltpu.VMEM((NS, RPT), jnp.float32),
            *([pltpu.VMEM((CHUNK, D_HID), jnp.float32)] * NBUF),
            pltpu.VMEM((RPT, D_HID), jnp.float32),
            pltpu.VMEM((RPT, D_HID), jnp.float32),
            pltpu.VMEM((RPT, D_HID), jnp.float32),
            pltpu.VMEM_SHARED((NS, N_PAD), jnp.float32),
            pltpu.VMEM_SHARED((N_PAD, D_HID), jnp.float32),
            pltpu.VMEM_SHARED((N_PAD, D_HID), jnp.float32),
            *([pltpu.SemaphoreType.DMA] * (2 * NBUF)),
            pltpu.SemaphoreType.DMA,
        ],
        compiler_params=cparams,
    )
    def sc_spmm1(h1_hbm, edge_hbm, zeros_hbm,
                 s1p_hbm, dinv_hbm,
                 sall, dall, dst2d, dalla, degacc, dred, *rest):
        rows = rest[0:NBUF]
        h1b, g1b, dvb, degsh, g1s, accs = rest[NBUF:NBUF + 6]
        gs = rest[NBUF + 6:NBUF + 6 + NBUF]
        ss = rest[NBUF + 6 + NBUF:NBUF + 6 + 2 * NBUF]
        ws = rest[NBUF + 6 + 2 * NBUF]
        cid = lax.axis_index("c")
        sid = lax.axis_index("s")
        wid = cid * NS + sid
        r0 = sid * RPT
        dpt = E // NS
        ones16 = jnp.ones((D_HID,), jnp.float32)
        zero16 = jnp.zeros((D_HID,), jnp.float32)
        pltpu.sync_copy(h1_hbm.at[pl.ds(r0, RPT)], h1b)
        pltpu.sync_copy(edge_hbm.at[1, pl.ds(sid * dpt, dpt)], dalla)
        _load_slab(edge_hbm, 0, wid, sall)
        _load_slab(edge_hbm, 1, wid, dall)
        _build_2d(dall, dst2d)

        @pl.loop(0, N_PAD // 16)
        def _(i):
            degacc[pl.ds(i * 16, 16)] = zero16

        @pl.loop(0, dpt // 16)
        def _(k):
            idx = dalla[pl.ds(k * 16, 16)]
            plsc.addupdate_scatter(degacc, [idx], ones16)

        pltpu.sync_copy(degacc, degsh.at[sid])
        plsc.subcore_barrier()
        pltpu.sync_copy(degsh.at[pl.ds(0, NS), pl.ds(r0, RPT)], dred)

        @pl.loop(0, RPT // 16)
        def _(v):
            dsum = dred[0, pl.ds(v * 16, 16)]
            for r in range(1, NS):
                dsum = dsum + dred[r, pl.ds(v * 16, 16)]
            dv = _rsqrt16(dsum + 1.0)
            for l in range(16):
                bc = _bcast_lane(dv, l)
                dvb[v * 16 + l, :] = bc
                g1b[v * 16 + l, :] = bc * h1b[v * 16 + l, :]

        pltpu.sync_copy(g1b, g1s.at[pl.ds(r0, RPT)])

        @pl.when(cid == 0)
        def _():
            pltpu.sync_copy(g1b, accs.at[pl.ds(r0, RPT)])

        @pl.when(cid != 0)
        def _():
            pltpu.sync_copy(zeros_hbm.at[pl.ds(r0, RPT)],
                            accs.at[pl.ds(r0, RPT)])

        own = (sid < NS // 2) == (cid == 0)

        @pl.when(own)
        def _():
            pltpu.async_copy(dvb, dinv_hbm.at[pl.ds(r0, RPT)], ws)

        plsc.subcore_barrier()
        _edge_loop(g1s, sall, dst2d, rows, gs, ss, accs)
        plsc.subcore_barrier()
        pltpu.sync_copy(accs.at[pl.ds(r0, RPT)],
                        s1p_hbm.at[cid, pl.ds(r0, RPT)])

        @pl.when(own)
        def _():
            pltpu.make_async_copy(dvb, dinv_hbm.at[pl.ds(r0, RPT)], ws).wait()

    @functools.partial(
        pl.kernel,
        out_type=partials,
        mesh=mesh,
        scratch_types=[
            pltpu.VMEM((NCHUNK * CHUNK,), jnp.int32),
            pltpu.VMEM((NCHUNK * CHUNK,), jnp.int32),
            pltpu.VMEM((NCHUNK, CHUNK), jnp.int32),
            *([pltpu.VMEM((CHUNK, D_HID), jnp.float32)] * NBUF),
            pltpu.VMEM((RPT, D_HID), jnp.float32),
            pltpu.VMEM((RPT, D_HID), jnp.float32),
            pltpu.VMEM((RPT, D_HID), jnp.float32),
            pltpu.VMEM((D_HID,), jnp.float32),
            pltpu.VMEM_SHARED((N_PAD, D_HID), jnp.float32),
            pltpu.VMEM_SHARED((N_PAD, D_HID), jnp.float32),
            *([pltpu.SemaphoreType.DMA] * (2 * NBUF)),
        ],
        compiler_params=cparams,
    )
    def sc_spmm2(s1p_hbm, dinv_hbm, b1_hbm, edge_hbm, zeros_hbm, s2p_hbm,
                 sall, dall, dst2d, *rest):
        rows = rest[0:NBUF]
        p0b, p1b, dvb, b1v, us, accs = rest[NBUF:NBUF + 6]
        gs = rest[NBUF + 6:NBUF + 6 + NBUF]
        ss = rest[NBUF + 6 + NBUF:NBUF + 6 + 2 * NBUF]
        cid = lax.axis_index("c")
        sid = lax.axis_index("s")
        wid = cid * NS + sid
        r0 = sid * RPT
        pltpu.sync_copy(s1p_hbm.at[0, pl.ds(r0, RPT)], p0b)
        pltpu.sync_copy(s1p_hbm.at[1, pl.ds(r0, RPT)], p1b)
        pltpu.sync_copy(dinv_hbm.at[pl.ds(r0, RPT)], dvb)
        pltpu.sync_copy(b1_hbm, b1v)
        _load_slab(edge_hbm, 0, wid, sall)
        _load_slab(edge_hbm, 1, wid, dall)
        _build_2d(dall, dst2d)
        b1r = b1v[...]

        @pl.loop(0, RPT)
        def _(i):
            dv = dvb[i, :]
            z = jnp.maximum(dv * (p0b[i, :] + p1b[i, :]) + b1r, 0.0)
            p0b[i, :] = dv * z

        pltpu.sync_copy(p0b, us.at[pl.ds(r0, RPT)])

        @pl.when(cid == 0)
        def _():
            pltpu.sync_copy(p0b, accs.at[pl.ds(r0, RPT)])

        @pl.when(cid != 0)
        def _():
            pltpu.sync_copy(zeros_hbm.at[pl.ds(r0, RPT)],
                            accs.at[pl.ds(r0, RPT)])

        plsc.subcore_barrier()
        _edge_loop(us, sall, dst2d, rows, gs, ss, accs)
        plsc.subcore_barrier()

        pltpu.sync_copy(accs.at[pl.ds(r0, RPT)], p1b)

        @pl.loop(0, RPT)
        def _(i):
            p1b[i, :] = p1b[i, :] * dvb[i, :]

        pltpu.sync_copy(p1b, s2p_hbm.at[cid, pl.ds(r0, RPT)])

    return sc_spmm1, sc_spmm2


def _tc_mm_body(x_ref, w_ref, o_ref):
    h = jnp.dot(x_ref[...].astype(jnp.bfloat16),
                w_ref[...].astype(jnp.bfloat16),
                preferred_element_type=jnp.float32)
    o_ref[...] = jnp.pad(h, ((0, N_PAD - N), (0, 0)))


def _tc_final_body(s2p_ref, w2_ref, b2_ref, o_ref):
    y = s2p_ref[0] + s2p_ref[1]
    w2p = jnp.pad(w2_ref[...], ((0, 0), (0, D_HID - N_CLS)))
    b2p = jnp.pad(b2_ref[...], (0, D_HID - N_CLS), constant_values=-1e30)
    logits = jnp.dot(y, w2p, preferred_element_type=jnp.float32) + b2p
    m = jnp.max(logits, axis=1, keepdims=True)
    e = jnp.exp(logits - m)
    lse = m + jnp.log(jnp.sum(e, axis=1, keepdims=True))
    o_ref[...] = (logits - lse)[:N, :N_CLS]


def _f32(shape):
    return jax.ShapeDtypeStruct(shape, jnp.float32)


def kernel(x, edge_index, W1, b1, W2, b2):
    sc_spmm1, sc_spmm2 = _sc_kernels()
    zeros = jnp.zeros((N_PAD, D_HID), jnp.float32)

    h1 = pl.pallas_call(_tc_mm_body, out_shape=_f32((N_PAD, D_HID)))(x, W1)
    s1p, dinvb = sc_spmm1(h1, edge_index, zeros)
    s2p = sc_spmm2(s1p, dinvb, b1, edge_index, zeros)
    out = pl.pallas_call(_tc_final_body, out_shape=_f32((N, N_CLS)))(
        s2p, W2, b2)
    return out


# speedup vs baseline: 43.2519x; 1.0657x over previous
import functools

import jax
import jax.numpy as jnp
from jax import lax
from jax.experimental import pallas as pl
from jax.experimental.pallas import tpu as pltpu
from jax.experimental.pallas import tpu_sc as plsc

N = 10000
E = 160000
D_IN = 256
D_HID = 16
N_CLS = 3

NC = 2
NS = 16
NW = NC * NS
CHUNK = 128
N_PAD = 10240
EPW = E // NW
NCHUNK = 40
RPT = N_PAD // NS
DUMP = N_PAD - 1
TAIL = EPW - (NCHUNK - 1) * CHUNK


_GATHER_DNUMS = lax.GatherDimensionNumbers(
    offset_dims=(), collapsed_slice_dims=(0,), start_index_map=(0,))


def _bcast_lane(v, l):
    idx = jnp.full((16, 1), l, jnp.int32)
    return lax.gather(v, idx, _GATHER_DNUMS, (1,),
                      mode=lax.GatherScatterMode.PROMISE_IN_BOUNDS)


def _rsqrt16(x):
    i = plsc.bitcast(x, jnp.int32)
    i = jnp.int32(0x5F3759DF) - lax.shift_right_logical(i, 1)
    y = plsc.bitcast(i, jnp.float32)
    for _ in range(3):
        y = y * (1.5 - 0.5 * x * y * y)
    return y


def _load_slab(edge_hbm, row, wid, buf):
    pltpu.sync_copy(edge_hbm.at[row, pl.ds(wid * EPW, EPW)],
                    buf.at[pl.ds(0, EPW)])
    lanes = lax.iota(jnp.int32, 16)
    first = (NCHUNK - 1) * CHUNK + (TAIL // 16) * 16
    real = buf[pl.ds(first, 16)]
    buf[pl.ds(first, 16)] = jnp.where(lanes < (EPW - first),
                                      real, jnp.int32(DUMP))
    for k in range(first + 16, NCHUNK * CHUNK, 16):
        buf[pl.ds(k, 16)] = jnp.full((16,), DUMP, jnp.int32)


def _build_2d(buf, idx2d):
    @pl.loop(0, NCHUNK)
    def _(j):
        for t in range(CHUNK // 16):
            idx2d[j, pl.ds(t * 16, 16)] = buf[pl.ds(j * CHUNK + t * 16, 16)]


NBUF = 4


def _edge_loop(src_spmem, sall, dst2d, rows, gs, ss, acc):
    def gref(j):
        return src_spmem.at[sall.at[pl.ds(j * CHUNK, CHUNK)]]

    for b in range(NBUF):
        pltpu.async_copy(gref(b), rows[b], gs[b])

    @pl.loop(0, NCHUNK, step=NBUF)
    def _(j):
        for b in range(NBUF):
            pltpu.make_async_copy(gref(j + b), rows[b], gs[b]).wait()
            pltpu.async_copy(rows[b], acc.at[dst2d.at[j + b]], ss[b], add=True)
        for b in range(NBUF):
            pltpu.make_async_copy(rows[b], acc.at[dst2d.at[j + b]],
                                  ss[b]).wait()

            @pl.when(j + NBUF + b < NCHUNK)
            def _():
                pltpu.async_copy(gref(j + NBUF + b), rows[b], gs[b])


@functools.cache
def _sc_kernels():
    mesh = plsc.VectorSubcoreMesh(core_axis_name="c", subcore_axis_name="s")
    partials = jax.ShapeDtypeStruct((NC, N_PAD, D_HID), jnp.float32)
    full = jax.ShapeDtypeStruct((N_PAD, D_HID), jnp.float32)
    cparams = pltpu.CompilerParams(use_tc_tiling_on_sc=False,
                                   needs_layout_passes=False)

    @functools.partial(
        pl.kernel,
        out_type=full,
        mesh=mesh,
        scratch_types=[
            pltpu.VMEM((E // NS,), jnp.int32),
            pltpu.VMEM((N_PAD,), jnp.float32),
            pltpu.VMEM((NS, RPT), jnp.float32),
            pltpu.VMEM((RPT, D_HID), jnp.float32),
            pltpu.VMEM_SHARED((NS, N_PAD), jnp.float32),
        ],
        compiler_params=cparams,
    )
    def sc_dinv(edge_hbm, dinv_hbm, dalla, degacc, dred, dvb, degsh):
        cid = lax.axis_index("c")
        sid = lax.axis_index("s")
        r0 = sid * RPT
        dpt = E // NS
        ones16 = jnp.ones((D_HID,), jnp.float32)
        zero16 = jnp.zeros((D_HID,), jnp.float32)
        pltpu.sync_copy(edge_hbm.at[1, pl.ds(sid * dpt, dpt)], dalla)

        @pl.loop(0, N_PAD // 16)
        def _(i):
            degacc[pl.ds(i * 16, 16)] = zero16

        @pl.loop(0, dpt // 16)
        def _(k):
            idx = dalla[pl.ds(k * 16, 16)]
            plsc.addupdate_scatter(degacc, [idx], ones16)

        pltpu.sync_copy(degacc, degsh.at[sid])
        plsc.subcore_barrier()
        pltpu.sync_copy(degsh.at[pl.ds(0, NS), pl.ds(r0, RPT)], dred)

        @pl.loop(0, RPT // 16)
        def _(v):
            dsum = dred[0, pl.ds(v * 16, 16)]
            for r in range(1, NS):
                dsum = dsum + dred[r, pl.ds(v * 16, 16)]
            dv = _rsqrt16(dsum + 1.0)
            for l in range(16):
                dvb[v * 16 + l, :] = _bcast_lane(dv, l)

        @pl.when((sid < NS // 2) == (cid == 0))
        def _():
            pltpu.sync_copy(dvb, dinv_hbm.at[pl.ds(r0, RPT)])

    @functools.partial(
        pl.kernel,
        out_type=partials,
        mesh=mesh,
        scratch_types=[
            pltpu.VMEM((NCHUNK * CHUNK,), jnp.int32),
            pltpu.VMEM((NCHUNK * CHUNK,), jnp.int32),
            pltpu.VMEM((NCHUNK, CHUNK), jnp.int32),
            *([pltpu.VMEM((CHUNK, D_HID), jnp.float32)] * NBUF),
            pltpu.VMEM((RPT, D_HID), jnp.float32),
            pltpu.VMEM((RPT, D_HID), jnp.float32),
            pltpu.VMEM((RPT, D_HID), jnp.float32),
            pltpu.VMEM_SHARED((N_PAD, D_HID), jnp.float32),
            pltpu.VMEM_SHARED((N_PAD, D_HID), jnp.float32),
            *([pltpu.SemaphoreType.DMA] * (2 * NBUF)),
        ],
        compiler_params=cparams,
    )
    def sc_spmm1(h1_hbm, dinv_hbm, edge_hbm, zeros_hbm, s1p_hbm,
                 sall, dall, dst2d, *rest):
        rows = rest[0:NBUF]
        h1b, g1b, dvb, g1s, accs = rest[NBUF:NBUF + 5]
        gs = rest[NBUF + 5:NBUF + 5 + NBUF]
        ss = rest[NBUF + 5 + NBUF:NBUF + 5 + 2 * NBUF]
        cid = lax.axis_index("c")
        sid = lax.axis_index("s")
        wid = cid * NS + sid
        r0 = sid * RPT
        pltpu.sync_copy(h1_hbm.at[pl.ds(r0, RPT)], h1b)
        pltpu.sync_copy(dinv_hbm.at[pl.ds(r0, RPT)], dvb)
        _load_slab(edge_hbm, 0, wid, sall)
        _load_slab(edge_hbm, 1, wid, dall)
        _build_2d(dall, dst2d)

        @pl.loop(0, RPT)
        def _(i):
            g1b[i, :] = dvb[i, :] * h1b[i, :]

        pltpu.sync_copy(g1b, g1s.at[pl.ds(r0, RPT)])

        @pl.when(cid == 0)
        def _():
            pltpu.sync_copy(g1b, accs.at[pl.ds(r0, RPT)])

        @pl.when(cid != 0)
        def _():
            pltpu.sync_copy(zeros_hbm.at[pl.ds(r0, RPT)],
                            accs.at[pl.ds(r0, RPT)])

        plsc.subcore_barrier()
        _edge_loop(g1s, sall, dst2d, rows, gs, ss, accs)
        plsc.subcore_barrier()
        pltpu.sync_copy(accs.at[pl.ds(r0, RPT)],
                        s1p_hbm.at[cid, pl.ds(r0, RPT)])

    @functools.partial(
        pl.kernel,
        out_type=partials,
        mesh=mesh,
        scratch_types=[
            pltpu.VMEM((NCHUNK * CHUNK,), jnp.int32),
            pltpu.VMEM((NCHUNK * CHUNK,), jnp.int32),
            pltpu.VMEM((NCHUNK, CHUNK), jnp.int32),
            *([pltpu.VMEM((CHUNK, D_HID), jnp.float32)] * NBUF),
            pltpu.VMEM((RPT, D_HID), jnp.float32),
            pltpu.VMEM((RPT, D_HID), jnp.float32),
            pltpu.VMEM((RPT, D_HID), jnp.float32),
            pltpu.VMEM((D_HID,), jnp.float32),
            pltpu.VMEM_SHARED((N_PAD, D_HID), jnp.float32),
            pltpu.VMEM_SHARED((N_PAD, D_HID), jnp.float32),
            *([pltpu.SemaphoreType.DMA] * (2 * NBUF)),
        ],
        compiler_params=cparams,
    )
    def sc_spmm2(s1p_hbm, dinv_hbm, b1_hbm, edge_hbm, zeros_hbm, s2p_hbm,
                 sall, dall, dst2d, *rest):
        rows = rest[0:NBUF]
        p0b, p1b, dvb, b1v, us, accs = rest[NBUF:NBUF + 6]
        gs = rest[NBUF + 6:NBUF + 6 + NBUF]
        ss = rest[NBUF + 6 + NBUF:NBUF + 6 + 2 * NBUF]
        cid = lax.axis_index("c")
        sid = lax.axis_index("s")
        wid = cid * NS + sid
        r0 = sid * RPT
        pltpu.sync_copy(s1p_hbm.at[0, pl.ds(r0, RPT)], p0b)
        pltpu.sync_copy(s1p_hbm.at[1, pl.ds(r0, RPT)], p1b)
        pltpu.sync_copy(dinv_hbm.at[pl.ds(r0, RPT)], dvb)
        pltpu.sync_copy(b1_hbm, b1v)
        _load_slab(edge_hbm, 0, wid, sall)
        _load_slab(edge_hbm, 1, wid, dall)
        _build_2d(dall, dst2d)
        b1r = b1v[...]

        @pl.loop(0, RPT)
        def _(i):
            dv = dvb[i, :]
            z = jnp.maximum(dv * (p0b[i, :] + p1b[i, :]) + b1r, 0.0)
            p0b[i, :] = dv * z

        pltpu.sync_copy(p0b, us.at[pl.ds(r0, RPT)])

        @pl.when(cid == 0)
        def _():
            pltpu.sync_copy(p0b, accs.at[pl.ds(r0, RPT)])

        @pl.when(cid != 0)
        def _():
            pltpu.sync_copy(zeros_hbm.at[pl.ds(r0, RPT)],
                            accs.at[pl.ds(r0, RPT)])

        plsc.subcore_barrier()
        _edge_loop(us, sall, dst2d, rows, gs, ss, accs)
        plsc.subcore_barrier()

        pltpu.sync_copy(accs.at[pl.ds(r0, RPT)], p1b)

        @pl.loop(0, RPT)
        def _(i):
            p1b[i, :] = p1b[i, :] * dvb[i, :]

        pltpu.sync_copy(p1b, s2p_hbm.at[cid, pl.ds(r0, RPT)])

    return sc_dinv, sc_spmm1, sc_spmm2


def _tc_mm_body(x_ref, w_ref, o_ref):
    h = jnp.dot(x_ref[...].astype(jnp.bfloat16),
                w_ref[...].astype(jnp.bfloat16),
                preferred_element_type=jnp.float32)
    o_ref[...] = jnp.pad(h, ((0, N_PAD - N), (0, 0)))


def _tc_final_body(s2p_ref, w2_ref, b2_ref, o_ref):
    y = s2p_ref[0] + s2p_ref[1]
    w2p = jnp.pad(w2_ref[...], ((0, 0), (0, D_HID - N_CLS)))
    b2p = jnp.pad(b2_ref[...], (0, D_HID - N_CLS), constant_values=-1e30)
    logits = jnp.dot(y, w2p, preferred_element_type=jnp.float32) + b2p
    m = jnp.max(logits, axis=1, keepdims=True)
    e = jnp.exp(logits - m)
    lse = m + jnp.log(jnp.sum(e, axis=1, keepdims=True))
    o_ref[...] = (logits - lse)[:N, :N_CLS]


def _f32(shape):
    return jax.ShapeDtypeStruct(shape, jnp.float32)


def kernel(x, edge_index, W1, b1, W2, b2):
    sc_dinv, sc_spmm1, sc_spmm2 = _sc_kernels()
    zeros = jnp.zeros((N_PAD, D_HID), jnp.float32)

    h1 = pl.pallas_call(_tc_mm_body, out_shape=_f32((N_PAD, D_HID)))(x, W1)
    dinvb = sc_dinv(edge_index)
    s1p = sc_spmm1(h1, dinvb, edge_index, zeros)
    s2p = sc_spmm2(s1p, dinvb, b1, edge_index, zeros)
    out = pl.pallas_call(_tc_final_body, out_shape=_f32((N, N_CLS)))(
        s2p, W2, b2)
    return out
